# Initial kernel scaffold; baseline (speedup 1.0000x reference)
import jax
import jax.numpy as jnp
from jax.experimental import pallas as pl


def kernel(x, W1, a_src1, a_dst1, b1, W2, a_src2, a_dst2, b2, src0, dst0, src1, dst1):
    raise NotImplementedError("write your pallas kernel here")


# speedup vs baseline: 11.8520x; 11.8520x over previous
import functools
import jax
import jax.numpy as jnp
from jax import lax
from jax.experimental import pallas as pl

_N0, _N1, _N2 = 100000, 40000, 4096
_E0, _E1 = 600000, 40960
_IN, _H, _C1, _C2 = 128, 4, 32, 64


def _mm1_body(x_ref, w_ref, a_ref, hs_ref, al_ref):
    hs = jnp.dot(x_ref[:], w_ref[:], preferred_element_type=jnp.float32)
    hs_ref[:] = hs
    al_ref[:] = jnp.dot(hs, a_ref[:], preferred_element_type=jnp.float32)


def _mm1(x, w, a, tile):
    n = x.shape[0]
    grid = n // tile
    return pl.pallas_call(
        _mm1_body,
        grid=(grid,),
        in_specs=[
            pl.BlockSpec((tile, x.shape[1]), lambda i: (i, 0)),
            pl.BlockSpec(w.shape, lambda i: (0, 0)),
            pl.BlockSpec(a.shape, lambda i: (0, 0)),
        ],
        out_specs=[
            pl.BlockSpec((tile, w.shape[1]), lambda i: (i, 0)),
            pl.BlockSpec((tile, a.shape[1]), lambda i: (i, 0)),
        ],
        out_shape=[
            jax.ShapeDtypeStruct((n, w.shape[1]), jnp.float32),
            jax.ShapeDtypeStruct((n, a.shape[1]), jnp.float32),
        ],
    )(x, w, a)


def _mm2_body(num_ref, den_ref, b_ref, w_ref, a_ref, hs_ref, al_ref):
    den = jnp.repeat(den_ref[:, :_H], _C1, axis=1)
    h = num_ref[:] / (den + 1e-16) + b_ref[:]
    h = jnp.where(h > 0, h, jnp.exp(jnp.minimum(h, 0.0)) - 1.0)
    hs = jnp.dot(h, w_ref[:], preferred_element_type=jnp.float32)
    hs_ref[:] = hs
    al_ref[:] = jnp.dot(hs, a_ref[:], preferred_element_type=jnp.float32)


def _mm2(numer, denom, b, w, a, tile):
    n = numer.shape[0]
    return pl.pallas_call(
        _mm2_body,
        grid=(n // tile,),
        in_specs=[
            pl.BlockSpec((tile, numer.shape[1]), lambda i: (i, 0)),
            pl.BlockSpec((tile, denom.shape[1]), lambda i: (i, 0)),
            pl.BlockSpec(b.shape, lambda i: (0, 0)),
            pl.BlockSpec(w.shape, lambda i: (0, 0)),
            pl.BlockSpec(a.shape, lambda i: (0, 0)),
        ],
        out_specs=[
            pl.BlockSpec((tile, w.shape[1]), lambda i: (i, 0)),
            pl.BlockSpec((tile, a.shape[1]), lambda i: (i, 0)),
        ],
        out_shape=[
            jax.ShapeDtypeStruct((n, w.shape[1]), jnp.float32),
            jax.ShapeDtypeStruct((n, a.shape[1]), jnp.float32),
        ],
    )(numer, denom, b, w, a)


def _fin_body(num_ref, den_ref, b_ref, out_ref):
    t = num_ref.shape[0]
    x4 = num_ref[:].reshape(t, _H, _C2)
    den = den_ref[:, :_H].reshape(t, _H, 1)
    m = jnp.mean(x4 / (den + 1e-16), axis=1) + b_ref[:]
    z = m - jnp.max(m, axis=-1, keepdims=True)
    out_ref[:] = z - jnp.log(jnp.sum(jnp.exp(z), axis=-1, keepdims=True))


def _fin(numer, denom, b, tile):
    n = numer.shape[0]
    return pl.pallas_call(
        _fin_body,
        grid=(n // tile,),
        in_specs=[
            pl.BlockSpec((tile, numer.shape[1]), lambda i: (i, 0)),
            pl.BlockSpec((tile, denom.shape[1]), lambda i: (i, 0)),
            pl.BlockSpec(b.shape, lambda i: (0, 0)),
        ],
        out_specs=pl.BlockSpec((tile, _C2), lambda i: (i, 0)),
        out_shape=jax.ShapeDtypeStruct((n, _C2), jnp.float32),
    )(numer, denom, b)


def _edge_phase_xla(hs, asrc, adst, src, dst, n_dst, ch):
    alpha = asrc[src] + adst[dst]
    alpha = jnp.where(alpha >= 0, alpha, 0.2 * alpha)
    ex = jnp.exp(alpha)
    denom = jnp.zeros((n_dst, _H), jnp.float32).at[dst].add(ex)
    msg = hs[src] * jnp.repeat(ex, ch, axis=1)
    numer = jnp.zeros((n_dst, _H * ch), jnp.float32).at[dst].add(msg)
    denom = jnp.pad(denom, ((0, 0), (0, 16 - _H)))
    return numer, denom


def _alpha_mat(a_s, a_d):
    eye = jnp.eye(_H, 16, dtype=jnp.float32)
    As = jnp.einsum("hc,hk->hck", a_s, eye).reshape(-1, 16)
    Ad = jnp.einsum("hc,hk->hck", a_d, eye).reshape(-1, 16)
    return jnp.concatenate([As, Ad], axis=1)


def kernel(x, W1, a_src1, a_dst1, b1, W2, a_src2, a_dst2, b2, src0, dst0, src1, dst1):
    A1 = _alpha_mat(a_src1, a_dst1)
    A2 = _alpha_mat(a_src2, a_dst2)
    hs1, al1 = _mm1(x, W1, A1, 1000)
    numer1, denom1 = _edge_phase_xla(
        hs1, al1[:, :_H], al1[:_N1, 16:16 + _H], src0, dst0, _N1, _C1)
    hs2, al2 = _mm2(numer1, denom1, b1.reshape(1, -1), W2, A2, 1000)
    numer2, denom2 = _edge_phase_xla(
        hs2, al2[:, :_H], al2[:_N2, 16:16 + _H], src1, dst1, _N2, _C2)
    return _fin(numer2, denom2, b2.reshape(1, -1), 512)


# speedup vs baseline: 12.6709x; 1.0691x over previous
import functools
import jax
import jax.numpy as jnp
from jax import lax
from jax.experimental import pallas as pl
from jax.experimental.pallas import tpu as pltpu
from jax.experimental.pallas import tpu_sc as plsc

_NC, _NS = 2, 16
_NW = _NC * _NS
_LANES = 16
_CH = 64

_N0, _N1, _N2 = 100000, 40000, 4096
_E0, _E1 = 600000, 40960
_IN, _H, _C1, _C2 = 128, 4, 32, 64


def _mm1_body(x_ref, w_ref, a_ref, hs_ref, al_ref):
    hs = jnp.dot(x_ref[:], w_ref[:], preferred_element_type=jnp.float32)
    hs_ref[:] = hs
    al_ref[:] = jnp.dot(hs, a_ref[:], preferred_element_type=jnp.float32)


def _mm1(x, w, a, tile):
    n = x.shape[0]
    grid = n // tile
    return pl.pallas_call(
        _mm1_body,
        grid=(grid,),
        in_specs=[
            pl.BlockSpec((tile, x.shape[1]), lambda i: (i, 0)),
            pl.BlockSpec(w.shape, lambda i: (0, 0)),
            pl.BlockSpec(a.shape, lambda i: (0, 0)),
        ],
        out_specs=[
            pl.BlockSpec((tile, w.shape[1]), lambda i: (i, 0)),
            pl.BlockSpec((tile, a.shape[1]), lambda i: (i, 0)),
        ],
        out_shape=[
            jax.ShapeDtypeStruct((n, w.shape[1]), jnp.float32),
            jax.ShapeDtypeStruct((n, a.shape[1]), jnp.float32),
        ],
    )(x, w, a)


def _mm2_body(num_ref, den_ref, b_ref, w_ref, a_ref, hs_ref, al_ref):
    den = jnp.repeat(den_ref[:, :_H], _C1, axis=1)
    h = num_ref[:] / (den + 1e-16) + b_ref[:]
    h = jnp.where(h > 0, h, jnp.exp(jnp.minimum(h, 0.0)) - 1.0)
    hs = jnp.dot(h, w_ref[:], preferred_element_type=jnp.float32)
    hs_ref[:] = hs
    al_ref[:] = jnp.dot(hs, a_ref[:], preferred_element_type=jnp.float32)


def _mm2(numer, denom, b, w, a, tile):
    n = numer.shape[0]
    return pl.pallas_call(
        _mm2_body,
        grid=(n // tile,),
        in_specs=[
            pl.BlockSpec((tile, numer.shape[1]), lambda i: (i, 0)),
            pl.BlockSpec((tile, denom.shape[1]), lambda i: (i, 0)),
            pl.BlockSpec(b.shape, lambda i: (0, 0)),
            pl.BlockSpec(w.shape, lambda i: (0, 0)),
            pl.BlockSpec(a.shape, lambda i: (0, 0)),
        ],
        out_specs=[
            pl.BlockSpec((tile, w.shape[1]), lambda i: (i, 0)),
            pl.BlockSpec((tile, a.shape[1]), lambda i: (i, 0)),
        ],
        out_shape=[
            jax.ShapeDtypeStruct((n, w.shape[1]), jnp.float32),
            jax.ShapeDtypeStruct((n, a.shape[1]), jnp.float32),
        ],
    )(numer, denom, b, w, a)


def _fin_body(num_ref, den_ref, b_ref, out_ref):
    t = num_ref.shape[0]
    x4 = num_ref[:].reshape(t, _H, _C2)
    den = den_ref[:, :_H].reshape(t, _H, 1)
    m = jnp.mean(x4 / (den + 1e-16), axis=1) + b_ref[:]
    z = m - jnp.max(m, axis=-1, keepdims=True)
    out_ref[:] = z - jnp.log(jnp.sum(jnp.exp(z), axis=-1, keepdims=True))


def _fin(numer, denom, b, tile):
    n = numer.shape[0]
    return pl.pallas_call(
        _fin_body,
        grid=(n // tile,),
        in_specs=[
            pl.BlockSpec((tile, numer.shape[1]), lambda i: (i, 0)),
            pl.BlockSpec((tile, denom.shape[1]), lambda i: (i, 0)),
            pl.BlockSpec(b.shape, lambda i: (0, 0)),
        ],
        out_specs=pl.BlockSpec((tile, _C2), lambda i: (i, 0)),
        out_shape=jax.ShapeDtypeStruct((n, _C2), jnp.float32),
    )(numer, denom, b)


def _ex_body(a_ref, b_ref, o_ref):
    al = a_ref[:] + b_ref[:]
    o_ref[:] = jnp.exp(jnp.where(al >= 0, al, 0.2 * al))


def _ex_tc(asg, adg, tile):
    n = asg.shape[0]
    return pl.pallas_call(
        _ex_body,
        grid=(n // tile,),
        in_specs=[pl.BlockSpec((tile, 16), lambda i: (i, 0)),
                  pl.BlockSpec((tile, 16), lambda i: (i, 0))],
        out_specs=pl.BlockSpec((tile, 16), lambda i: (i, 0)),
        out_shape=jax.ShapeDtypeStruct((n, 16), jnp.float32),
    )(asg, adg)


def _make_edge_sc(E, D):
    nchunk = (E + _CH - 1) // _CH
    cpw = (nchunk + _NW - 1) // _NW
    nhv = D // _H // _LANES
    mesh = plsc.VectorSubcoreMesh(core_axis_name="c", subcore_axis_name="s")

    @functools.partial(
        pl.kernel, mesh=mesh,
        out_type=jax.ShapeDtypeStruct((E, D), jnp.float32),
        scratch_types=[
            pltpu.VMEM((_CH,), jnp.int32),
            pltpu.VMEM((_CH, 16), jnp.float32),
            pltpu.VMEM((_CH, D), jnp.float32),
            pltpu.SemaphoreType.DMA,
        ],
    )
    def edge_kernel(src_h, ex_h, hs_h, out_h, src_v, exrows, hsrows, sem0):
        c = lax.axis_index("c")
        s = lax.axis_index("s")
        wid = s * _NC + c

        def chunk_body(i, _):
            base = jnp.minimum(i * _CH, E - _CH)
            pltpu.sync_copy(src_h.at[pl.ds(base, _CH)], src_v)
            g = pltpu.async_copy(hs_h.at[src_v], hsrows, sem0)
            pltpu.sync_copy(ex_h.at[pl.ds(base, _CH)], exrows)
            g.wait()

            def edge_body(e, _):
                exv = exrows[e, pl.ds(0, _LANES)]
                for h in range(_H):
                    bv = exv.at[jnp.full((_LANES,), h, jnp.int32)].get(
                        mode="promise_in_bounds")
                    for jj in range(nhv):
                        cs = pl.ds((h * nhv + jj) * _LANES, _LANES)
                        hsrows[e, cs] = hsrows[e, cs] * bv
                return 0
            lax.fori_loop(0, _CH, edge_body, 0)
            pltpu.sync_copy(hsrows, out_h.at[pl.ds(base, _CH)])
            return 0

        lax.fori_loop(jnp.minimum(wid * cpw, nchunk),
                      jnp.minimum((wid + 1) * cpw, nchunk),
                      chunk_body, 0)

    return edge_kernel


def _alpha_mat(a_s, a_d):
    eye = jnp.eye(_H, 16, dtype=jnp.float32)
    As = jnp.einsum("hc,hk->hck", a_s, eye).reshape(-1, 16)
    Ad = jnp.einsum("hc,hk->hck", a_d, eye).reshape(-1, 16)
    return jnp.concatenate([As, Ad], axis=1)


def kernel(x, W1, a_src1, a_dst1, b1, W2, a_src2, a_dst2, b2, src0, dst0, src1, dst1):
    A1 = _alpha_mat(a_src1, a_dst1)
    A2 = _alpha_mat(a_src2, a_dst2)
    hs1, al1 = _mm1(x, W1, A1, 1000)
    ex1 = _ex_tc(jnp.take(al1[:, :16], src0, axis=0),
                 jnp.take(al1[:_N1, 16:], dst0, axis=0), 1000)
    msg1 = _make_edge_sc(_E0, _H * _C1)(src0, ex1, hs1)
    numer1 = jnp.zeros((_N1, _H * _C1), jnp.float32).at[dst0].add(msg1)
    denom1 = jnp.pad(jnp.zeros((_N1, _H), jnp.float32).at[dst0].add(
        ex1[:, :_H]), ((0, 0), (0, 16 - _H)))
    hs2, al2 = _mm2(numer1, denom1, b1.reshape(1, -1), W2, A2, 1000)
    ex2 = _ex_tc(jnp.take(al2[:, :16], src1, axis=0),
                 jnp.take(al2[:_N2, 16:], dst1, axis=0), 1024)
    msg2 = _make_edge_sc(_E1, _H * _C2)(src1, ex2, hs2)
    numer2 = jnp.zeros((_N2, _H * _C2), jnp.float32).at[dst1].add(msg2)
    denom2 = jnp.pad(jnp.zeros((_N2, _H), jnp.float32).at[dst1].add(
        ex2[:, :_H]), ((0, 0), (0, 16 - _H)))
    return _fin(numer2, denom2, b2.reshape(1, -1), 512)


# speedup vs baseline: 12.7044x; 1.0026x over previous
import functools
import jax
import jax.numpy as jnp
from jax import lax
from jax.experimental import pallas as pl
from jax.experimental.pallas import tpu as pltpu
from jax.experimental.pallas import tpu_sc as plsc

_NC, _NS = 2, 16
_NW = _NC * _NS
_LANES = 16
_CH = 128

_N0, _N1, _N2 = 100000, 40000, 4096
_E0, _E1 = 600000, 40960
_IN, _H, _C1, _C2 = 128, 4, 32, 64


def _mm1_body(x_ref, w_ref, a_ref, hs_ref, al_ref):
    hs = jnp.dot(x_ref[:], w_ref[:], preferred_element_type=jnp.float32)
    hs_ref[:] = hs
    al_ref[:] = jnp.dot(hs, a_ref[:], preferred_element_type=jnp.float32)


def _mm1(x, w, a, tile):
    n = x.shape[0]
    grid = n // tile
    return pl.pallas_call(
        _mm1_body,
        grid=(grid,),
        in_specs=[
            pl.BlockSpec((tile, x.shape[1]), lambda i: (i, 0)),
            pl.BlockSpec(w.shape, lambda i: (0, 0)),
            pl.BlockSpec(a.shape, lambda i: (0, 0)),
        ],
        out_specs=[
            pl.BlockSpec((tile, w.shape[1]), lambda i: (i, 0)),
            pl.BlockSpec((tile, a.shape[1]), lambda i: (i, 0)),
        ],
        out_shape=[
            jax.ShapeDtypeStruct((n, w.shape[1]), jnp.float32),
            jax.ShapeDtypeStruct((n, a.shape[1]), jnp.float32),
        ],
    )(x, w, a)


def _mm2_body(num_ref, den_ref, b_ref, w_ref, a_ref, hs_ref, al_ref):
    den = jnp.repeat(den_ref[:, :_H], _C1, axis=1)
    h = num_ref[:] / (den + 1e-16) + b_ref[:]
    h = jnp.where(h > 0, h, jnp.exp(jnp.minimum(h, 0.0)) - 1.0)
    hs = jnp.dot(h, w_ref[:], preferred_element_type=jnp.float32)
    hs_ref[:] = hs
    al_ref[:] = jnp.dot(hs, a_ref[:], preferred_element_type=jnp.float32)


def _mm2(numer, denom, b, w, a, tile):
    n = numer.shape[0]
    return pl.pallas_call(
        _mm2_body,
        grid=(n // tile,),
        in_specs=[
            pl.BlockSpec((tile, numer.shape[1]), lambda i: (i, 0)),
            pl.BlockSpec((tile, denom.shape[1]), lambda i: (i, 0)),
            pl.BlockSpec(b.shape, lambda i: (0, 0)),
            pl.BlockSpec(w.shape, lambda i: (0, 0)),
            pl.BlockSpec(a.shape, lambda i: (0, 0)),
        ],
        out_specs=[
            pl.BlockSpec((tile, w.shape[1]), lambda i: (i, 0)),
            pl.BlockSpec((tile, a.shape[1]), lambda i: (i, 0)),
        ],
        out_shape=[
            jax.ShapeDtypeStruct((n, w.shape[1]), jnp.float32),
            jax.ShapeDtypeStruct((n, a.shape[1]), jnp.float32),
        ],
    )(numer, denom, b, w, a)


def _fin_body(num_ref, den_ref, b_ref, out_ref):
    t = num_ref.shape[0]
    x4 = num_ref[:].reshape(t, _H, _C2)
    den = den_ref[:, :_H].reshape(t, _H, 1)
    m = jnp.mean(x4 / (den + 1e-16), axis=1) + b_ref[:]
    z = m - jnp.max(m, axis=-1, keepdims=True)
    out_ref[:] = z - jnp.log(jnp.sum(jnp.exp(z), axis=-1, keepdims=True))


def _fin(numer, denom, b, tile):
    n = numer.shape[0]
    return pl.pallas_call(
        _fin_body,
        grid=(n // tile,),
        in_specs=[
            pl.BlockSpec((tile, numer.shape[1]), lambda i: (i, 0)),
            pl.BlockSpec((tile, denom.shape[1]), lambda i: (i, 0)),
            pl.BlockSpec(b.shape, lambda i: (0, 0)),
        ],
        out_specs=pl.BlockSpec((tile, _C2), lambda i: (i, 0)),
        out_shape=jax.ShapeDtypeStruct((n, _C2), jnp.float32),
    )(numer, denom, b)


def _ex_body(a_ref, b_ref, o_ref):
    al = a_ref[:] + b_ref[:]
    o_ref[:] = jnp.exp(jnp.where(al >= 0, al, 0.2 * al))


def _ex_tc(asg, adg, tile):
    n = asg.shape[0]
    return pl.pallas_call(
        _ex_body,
        grid=(n // tile,),
        in_specs=[pl.BlockSpec((tile, 16), lambda i: (i, 0)),
                  pl.BlockSpec((tile, 16), lambda i: (i, 0))],
        out_specs=pl.BlockSpec((tile, 16), lambda i: (i, 0)),
        out_shape=jax.ShapeDtypeStruct((n, 16), jnp.float32),
    )(asg, adg)


def _make_edge_sc(E, D):
    nchunk = (E + _CH - 1) // _CH
    cpw = (nchunk + _NW - 1) // _NW
    nhv = D // _H // _LANES
    mesh = plsc.VectorSubcoreMesh(core_axis_name="c", subcore_axis_name="s")

    @functools.partial(
        pl.kernel, mesh=mesh,
        out_type=jax.ShapeDtypeStruct((E, D), jnp.float32),
        scratch_types=[
            pltpu.VMEM((_CH,), jnp.int32),
            pltpu.VMEM((_CH, 16), jnp.float32),
            pltpu.VMEM((_CH, D), jnp.float32),
            pltpu.SemaphoreType.DMA,
        ],
    )
    def edge_kernel(src_h, ex_h, hs_h, out_h, src_v, exrows, hsrows, sem0):
        c = lax.axis_index("c")
        s = lax.axis_index("s")
        wid = s * _NC + c

        def chunk_body(i, _):
            base = jnp.minimum(i * _CH, E - _CH)
            pltpu.sync_copy(src_h.at[pl.ds(base, _CH)], src_v)
            g = pltpu.async_copy(hs_h.at[src_v], hsrows, sem0)
            pltpu.sync_copy(ex_h.at[pl.ds(base, _CH)], exrows)
            g.wait()

            def edge_body(e, _):
                exv = exrows[e, pl.ds(0, _LANES)]
                for h in range(_H):
                    bv = exv.at[jnp.full((_LANES,), h, jnp.int32)].get(
                        mode="promise_in_bounds")
                    for jj in range(nhv):
                        cs = pl.ds((h * nhv + jj) * _LANES, _LANES)
                        hsrows[e, cs] = hsrows[e, cs] * bv
                return 0
            lax.fori_loop(0, _CH, edge_body, 0)
            pltpu.sync_copy(hsrows, out_h.at[pl.ds(base, _CH)])
            return 0

        lax.fori_loop(jnp.minimum(wid * cpw, nchunk),
                      jnp.minimum((wid + 1) * cpw, nchunk),
                      chunk_body, 0)

    return edge_kernel


def _alpha_mat(a_s, a_d):
    eye = jnp.eye(_H, 16, dtype=jnp.float32)
    As = jnp.einsum("hc,hk->hck", a_s, eye).reshape(-1, 16)
    Ad = jnp.einsum("hc,hk->hck", a_d, eye).reshape(-1, 16)
    return jnp.concatenate([As, Ad], axis=1)


def kernel(x, W1, a_src1, a_dst1, b1, W2, a_src2, a_dst2, b2, src0, dst0, src1, dst1):
    A1 = _alpha_mat(a_src1, a_dst1)
    A2 = _alpha_mat(a_src2, a_dst2)
    hs1, al1 = _mm1(x, W1, A1, 1000)
    ex1 = _ex_tc(jnp.take(al1[:, :16], src0, axis=0),
                 jnp.take(al1[:_N1, 16:], dst0, axis=0), 1000)
    msg1 = _make_edge_sc(_E0, _H * _C1)(src0, ex1, hs1)
    numer1 = jnp.zeros((_N1, _H * _C1), jnp.float32).at[dst0].add(msg1)
    denom1 = jnp.pad(jnp.zeros((_N1, _H), jnp.float32).at[dst0].add(
        ex1[:, :_H]), ((0, 0), (0, 16 - _H)))
    hs2, al2 = _mm2(numer1, denom1, b1.reshape(1, -1), W2, A2, 1000)
    ex2 = _ex_tc(jnp.take(al2[:, :16], src1, axis=0),
                 jnp.take(al2[:_N2, 16:], dst1, axis=0), 1024)
    msg2 = _make_edge_sc(_E1, _H * _C2)(src1, ex2, hs2)
    numer2 = jnp.zeros((_N2, _H * _C2), jnp.float32).at[dst1].add(msg2)
    denom2 = jnp.pad(jnp.zeros((_N2, _H), jnp.float32).at[dst1].add(
        ex2[:, :_H]), ((0, 0), (0, 16 - _H)))
    return _fin(numer2, denom2, b2.reshape(1, -1), 512)
